# baseline (speedup 1.0000x reference)
import jax
import jax.numpy as jnp
from jax import lax
from jax.experimental import pallas as pl
from jax.experimental.pallas import tpu as pltpu
from jax.experimental.pallas import tpu_sc as plsc

_NC, _NS, _L = 2, 16, 16
_NW = _NC * _NS
_B, _T, _C = 8, 512, 1024
_LOUT = 2048
_QPB = _NW // _B
_QT = _LOUT // _QPB
_R = 32
_WR = 40
_NB = 3
_NCH = _QT // _R


def _body(x_hbm, dur_hbm, tl_hbm, out_hbm,
          tl_v, dur_v, hist_v, wbuf_v, zrow_v,
          gsem, ssem, msem):
    idx_v = hist_v
    wid = lax.axis_index("s") * _NC + lax.axis_index("c")
    b = wid // _QPB
    q = wid - b * _QPB
    brow0 = b * _LOUT

    tl_cp = pltpu.make_async_copy(tl_hbm, tl_v, msem.at[0])
    tl_cp.start()
    dur_cp = pltpu.make_async_copy(dur_hbm.at[b], dur_v, msem.at[1])
    dur_cp.start()

    zeros_i = jnp.zeros((_L,), jnp.int32)
    ones_i = jnp.ones((_L,), jnp.int32)
    zeros_f = jnp.zeros((_L,), jnp.float32)

    def _zero_hist(i, c):
        for u in range(4):
            hist_v[pl.ds(i * 4 * _L + u * _L, _L)] = zeros_i
        return c
    lax.fori_loop(0, _LOUT // _L // 4, _zero_hist, 0)

    def _zero_zrow(i, c):
        for u in range(4):
            zrow_v[0, pl.ds(i * 4 * _L + u * _L, _L)] = zeros_f
        return c
    lax.fori_loop(0, _C // _L // 4, _zero_zrow, 0)

    dur_cp.wait()
    tl_cp.wait()
    tl = jnp.max(tl_v[...])

    def _csum(i, carry):
        v = jnp.maximum(dur_v[pl.ds(i * _L, _L)], 1)
        s = plsc.cumsum(v) + carry
        plsc.addupdate_scatter(hist_v, [s], ones_i, mask=s < _LOUT)
        return jnp.max(s)
    total = lax.fori_loop(0, _T // _L, _csum, jnp.int32(0))
    limit = jnp.minimum(total, tl)

    base_row = b * _T
    def _psum(i, carry):
        ps = plsc.cumsum(hist_v[pl.ds(i * _L, _L)]) + carry
        hist_v[pl.ds(i * _L, _L)] = jnp.minimum(ps, _T - 1) + base_row
        return jnp.max(ps)
    psum_car = lax.fori_loop(0, 8, _psum, jnp.int32(0))

    def _t0(c):
        return (q + c * _QPB) * _R

    def _window(c):
        v0 = idx_v[pl.ds(_t0(c), _L)]
        vl = idx_v[pl.ds(_t0(c) + _R - _L, _L)]
        sw = pl.multiple_of(
            jnp.minimum((v0[0] // 8) * 8, _B * _T - _WR), 8)
        u4 = ((vl[15] - sw + 1 + 3) // 4) * 4
        return sw, u4

    def _win_dma(sw, u4, slot, do_wait):
        sw = pl.multiple_of(sw, 8)
        a32 = u4 >= 32
        r1 = jnp.where(a32, u4 - 32, u4)
        b16 = r1 >= 16
        r2 = jnp.where(b16, r1 - 16, r1)
        c8 = r2 >= 8
        r3 = jnp.where(c8, r2 - 8, r2)
        d4 = r3 >= 4
        off16 = pl.multiple_of(jnp.where(a32, _R, 0), 8)
        off8 = pl.multiple_of(off16 + jnp.where(b16, 16, 0), 8)
        off4 = pl.multiple_of(off8 + jnp.where(c8, 8, 0), 8)

        def _piece(pred, off, size):
            @pl.when(pred)
            def _():
                cp = pltpu.make_async_copy(
                    x_hbm.at[pl.ds(sw + off, size)],
                    wbuf_v.at[slot, pl.ds(off, size)],
                    gsem.at[slot])
                cp.wait() if do_wait else cp.start()

        _piece(a32, 0, _R)
        _piece(b16, off16, 16)
        _piece(c8, off8, 8)
        _piece(d4, off4, 4)

    def _store_wait(c):
        pltpu.make_async_copy(
            wbuf_v.at[c % _NB, pl.ds(0, _R)],
            out_hbm.at[pl.ds(brow0 + _t0(c), _R)],
            ssem.at[c % _NB]).wait()

    sw0, u40 = _window(0)
    _win_dma(sw0, u40, 0, False)
    lax.fori_loop(8, _LOUT // _L, _psum, psum_car)

    def _chunk(c, carry):
        sw, u4 = carry
        slot = c % _NB

        @pl.when(c >= _NB - 1)
        def _():
            _store_wait(c - (_NB - 1))

        swn, u4n = _window(jnp.minimum(c + 1, _NCH - 1))

        @pl.when(c + 1 < _NCH)
        def _():
            _win_dma(swn, u4n, (c + 1) % _NB, False)

        _win_dma(sw, u4, slot, True)

        t0c = _t0(c)
        out0 = brow0 + t0c
        for h in range(_R // _L):
            loc = idx_v[pl.ds(t0c + h * _L, _L)] - sw
            for r in range(_L):
                lr = loc[r]
                tpos = t0c + h * _L + r
                orow = out0 + h * _L + r

                @pl.when(tpos < limit)
                def _(lr=lr, orow=orow):
                    pltpu.make_async_copy(
                        wbuf_v.at[slot, pl.ds(lr, 1)],
                        out_hbm.at[pl.ds(orow, 1)],
                        ssem.at[slot]).start()

                @pl.when(tpos >= limit)
                def _(orow=orow):
                    pltpu.make_async_copy(
                        zrow_v,
                        out_hbm.at[pl.ds(orow, 1)],
                        ssem.at[slot]).start()
        return swn, u4n

    lax.fori_loop(0, _NCH, _chunk, (sw0, u40))

    def _drain(c, carry):
        _store_wait(c)
        return carry
    lax.fori_loop(_NCH - (_NB - 1), _NCH, _drain, 0)


_sc_call = pl.kernel(
    _body,
    out_type=jax.ShapeDtypeStruct((_B * _LOUT, _C), jnp.float32),
    mesh=plsc.VectorSubcoreMesh(core_axis_name="c", subcore_axis_name="s",
                                num_cores=_NC, num_subcores=_NS),
    compiler_params=pltpu.CompilerParams(needs_layout_passes=False),
    scratch_types=[
        pltpu.VMEM((_L,), jnp.int32),
        pltpu.VMEM((_T,), jnp.int32),
        pltpu.VMEM((_LOUT,), jnp.int32),
        pltpu.VMEM((_NB, _WR, _C), jnp.float32),
        pltpu.VMEM((1, _C), jnp.float32),
        pltpu.SemaphoreType.DMA((_NB,)),
        pltpu.SemaphoreType.DMA((_NB,)),
        pltpu.SemaphoreType.DMA((2,)),
    ],
)


def kernel(x, durations, target_length):
    x2 = x.reshape(_B * _T, _C)
    dur = durations.astype(jnp.int32)
    tl = jnp.full((_L,), target_length, dtype=jnp.int32)
    out = _sc_call(x2, dur, tl)
    return out.reshape(_B, _LOUT, _C)
